# baseline (speedup 1.0000x reference)
import functools

import jax
import jax.numpy as jnp
from jax import lax
from jax.experimental import pallas as pl
from jax.experimental.pallas import tpu as pltpu
from jax.experimental.pallas import tpu_sc as plsc

N_CAT = 26
N_NUM = 13
VOCAB = 100000
D = 32
T = N_CAT + N_NUM
ROWS_PER_B = 2 * (T + 1)

NC = 2
NS = 16
NW = NC * NS

CAT_CHUNK = 128
NUM_CHUNK = 128
STATIC_GROUP = 2
STATIC_ROWS = 41


def _num_mlp_body(x_ref, w1_ref, b1_ref, w2t_ref, b2_ref, out_ref):
    x = x_ref[...]
    outs = []
    for n in range(N_NUM):
        h1 = x[:, n:n + 1] * w1_ref[n][None, :] + b1_ref[n][None, :]
        z = jax.lax.dot_general(
            h1, w2t_ref[n],
            dimension_numbers=(((1,), (0,)), ((), ())),
            preferred_element_type=jnp.float32,
        ) + b2_ref[n][None, :]
        outs.append(jnp.tanh(z))
    out_ref[...] = jnp.concatenate(outs, axis=1)


def _num_mlp(x_num, num_w1, num_b1, num_w2t, num_b2):
    B = x_num.shape[0]
    BB = 2048
    grid = (B // BB,)
    return pl.pallas_call(
        _num_mlp_body,
        grid=grid,
        in_specs=[
            pl.BlockSpec((BB, N_NUM), lambda i: (i, 0)),
            pl.BlockSpec((N_NUM, D), lambda i: (0, 0)),
            pl.BlockSpec((N_NUM, D), lambda i: (0, 0)),
            pl.BlockSpec((N_NUM, D, D), lambda i: (0, 0, 0)),
            pl.BlockSpec((N_NUM, D), lambda i: (0, 0)),
        ],
        out_specs=pl.BlockSpec((BB, N_NUM * D), lambda i: (i, 0)),
        out_shape=jax.ShapeDtypeStruct((B, N_NUM * D), jnp.float32),
    )(x_num, num_w1, num_b1, num_w2t, num_b2)


def _sc_body(B, table_ref, src_idx_ref, dst_cat_ref, num_ref, dst_num_ref,
             static_src_ref, dst_static_ref, out_ref,
             idx_s, idx_d, rows, static_rows, idx_d82, sem):
    per_w = B // NW
    n_cat_chunks = per_w * N_CAT // CAT_CHUNK
    n_num_chunks = per_w * N_NUM // NUM_CHUNK
    n_static_chunks = per_w // STATIC_GROUP
    static_len = STATIC_GROUP * STATIC_ROWS

    w = lax.axis_index("s") * NC + lax.axis_index("c")

    pltpu.sync_copy(static_src_ref, static_rows)

    @pl.loop(0, n_cat_chunks)
    def _cat(g):
        pltpu.sync_copy(src_idx_ref.at[w, g], idx_s)
        pltpu.sync_copy(dst_cat_ref.at[w, g], idx_d)
        pltpu.async_copy(table_ref.at[idx_s], rows, sem).wait()
        pltpu.async_copy(rows, out_ref.at[idx_d], sem).wait()

    num_base = w * per_w * N_NUM

    @pl.loop(0, n_num_chunks)
    def _num(g):
        pltpu.sync_copy(dst_num_ref.at[w, g], idx_d)
        pltpu.sync_copy(num_ref.at[pl.ds(num_base + g * NUM_CHUNK, NUM_CHUNK)],
                        rows)
        pltpu.async_copy(rows, out_ref.at[idx_d], sem).wait()

    @pl.loop(0, n_static_chunks)
    def _static(g):
        pltpu.sync_copy(dst_static_ref.at[w, g], idx_d82)
        pltpu.async_copy(static_rows, out_ref.at[idx_d82], sem).wait()


def _sc_assemble(B, table, src_idx, dst_cat, num_emb, dst_num, static_src,
                 dst_static):
    mesh = plsc.VectorSubcoreMesh(core_axis_name="c", subcore_axis_name="s")
    static_len = STATIC_GROUP * STATIC_ROWS
    kern = pl.kernel(
        functools.partial(_sc_body, B),
        out_type=jax.ShapeDtypeStruct((B * ROWS_PER_B, D), jnp.float32),
        mesh=mesh,
        scratch_types=[
            pltpu.VMEM((CAT_CHUNK,), jnp.int32),
            pltpu.VMEM((CAT_CHUNK,), jnp.int32),
            pltpu.VMEM((CAT_CHUNK, D), jnp.float32),
            pltpu.VMEM((static_len, D), jnp.float32),
            pltpu.VMEM((static_len,), jnp.int32),
            pltpu.SemaphoreType.DMA,
        ],
        compiler_params=pltpu.CompilerParams(use_tc_tiling_on_sc=False),
    )
    return kern(table, src_idx, dst_cat, num_emb, dst_num, static_src,
                dst_static)


def kernel(x_cat, x_num, cat_tables, num_w1, num_b1, num_w2, num_b2,
           pos_table, cls_token):
    B = x_cat.shape[0]
    per_w = B // NW

    table = cat_tables.reshape(N_CAT * VOCAB, D)
    b = jnp.arange(B, dtype=jnp.int32)
    f = jnp.arange(N_CAT, dtype=jnp.int32)
    j = jnp.arange(N_NUM, dtype=jnp.int32)

    src_idx = (x_cat + (f * VOCAB)[None, :]).reshape(
        NW, per_w * N_CAT // CAT_CHUNK, CAT_CHUNK)
    dst_cat = (ROWS_PER_B * b[:, None] + 2 * f[None, :]).reshape(
        NW, per_w * N_CAT // CAT_CHUNK, CAT_CHUNK)
    dst_num = (ROWS_PER_B * b[:, None] + 2 * N_CAT + 2 * j[None, :]).reshape(
        NW, per_w * N_NUM // NUM_CHUNK, NUM_CHUNK)

    offs_static = jnp.concatenate([
        2 * jnp.arange(T, dtype=jnp.int32) + 1,
        jnp.array([2 * T, 2 * T + 1], dtype=jnp.int32),
    ])
    dst_static = (ROWS_PER_B * b[:, None] + offs_static[None, :]).reshape(
        NW, per_w // STATIC_GROUP, STATIC_GROUP * STATIC_ROWS)

    cls = cls_token.reshape(2 * D)
    static_pat = jnp.concatenate(
        [pos_table, cls[:D][None, :], cls[D:][None, :]], axis=0)
    static_src = jnp.tile(static_pat, (STATIC_GROUP, 1))

    num_w2t = num_w2.transpose(0, 2, 1)

    num_emb = _num_mlp(x_num, num_w1, num_b1, num_w2t, num_b2)
    num_rows = num_emb.reshape(B * N_NUM, D)

    out_flat = _sc_assemble(B, table, src_idx, dst_cat, num_rows, dst_num,
                            static_src, dst_static)
    return out_flat.reshape(B, T + 1, 2 * D)


# speedup vs baseline: 1.2126x; 1.2126x over previous
import functools

import jax
import jax.numpy as jnp
from jax import lax
from jax.experimental import pallas as pl
from jax.experimental.pallas import tpu as pltpu
from jax.experimental.pallas import tpu_sc as plsc

N_CAT = 26
N_NUM = 13
VOCAB = 100000
D = 32
T = N_CAT + N_NUM
ROWS_PER_B = 2 * (T + 1)

NC = 2
NS = 16
NW = NC * NS

CAT_CHUNK = 128
NUM_CHUNK = 128
STATIC_GROUP = 2
STATIC_ROWS = 41


def _num_mlp_body(x_ref, w1_ref, b1_ref, w2t_ref, b2_ref, out_ref):
    x = x_ref[...]
    outs = []
    for n in range(N_NUM):
        h1 = x[:, n:n + 1] * w1_ref[n][None, :] + b1_ref[n][None, :]
        z = jax.lax.dot_general(
            h1, w2t_ref[n],
            dimension_numbers=(((1,), (0,)), ((), ())),
            preferred_element_type=jnp.float32,
        ) + b2_ref[n][None, :]
        outs.append(jnp.tanh(z))
    out_ref[...] = jnp.concatenate(outs, axis=1)


def _num_mlp(x_num, num_w1, num_b1, num_w2t, num_b2):
    B = x_num.shape[0]
    BB = 2048
    grid = (B // BB,)
    return pl.pallas_call(
        _num_mlp_body,
        grid=grid,
        in_specs=[
            pl.BlockSpec((BB, N_NUM), lambda i: (i, 0)),
            pl.BlockSpec((N_NUM, D), lambda i: (0, 0)),
            pl.BlockSpec((N_NUM, D), lambda i: (0, 0)),
            pl.BlockSpec((N_NUM, D, D), lambda i: (0, 0, 0)),
            pl.BlockSpec((N_NUM, D), lambda i: (0, 0)),
        ],
        out_specs=pl.BlockSpec((BB, N_NUM * D), lambda i: (i, 0)),
        out_shape=jax.ShapeDtypeStruct((B, N_NUM * D), jnp.float32),
    )(x_num, num_w1, num_b1, num_w2t, num_b2)


KSET = 4


def _pipelined_phase(n_chunks, k_per_set, rows_a, rows_b, gs_a, gs_b,
                     ss_a, ss_b, out_ref, wait_idx_row,
                     fire_load, fire_store):
    npairs = n_chunks // (2 * k_per_set)

    def wait_store(rows, k, sem):
        pltpu.make_async_copy(
            rows.at[pl.ds(k * CAT_CHUNK, CAT_CHUNK)],
            out_ref.at[wait_idx_row], sem).wait()

    @pl.loop(0, npairs)
    def _pair(i):
        base_a = (2 * i) * k_per_set
        base_b = (2 * i + 1) * k_per_set

        @pl.when(i > 0)
        def _():
            for k in range(k_per_set):
                wait_store(rows_a, k, ss_a)

        loads_a = [
            fire_load(base_a + k, rows_a.at[pl.ds(k * CAT_CHUNK, CAT_CHUNK)],
                      gs_a)
            for k in range(k_per_set)
        ]

        @pl.when(i > 0)
        def _():
            for k in range(k_per_set):
                wait_store(rows_b, k, ss_b)

        loads_b = [
            fire_load(base_b + k, rows_b.at[pl.ds(k * CAT_CHUNK, CAT_CHUNK)],
                      gs_b)
            for k in range(k_per_set)
        ]

        for k in range(k_per_set):
            loads_a[k].wait()
            fire_store(base_a + k, rows_a.at[pl.ds(k * CAT_CHUNK, CAT_CHUNK)],
                       ss_a)
        for k in range(k_per_set):
            loads_b[k].wait()
            fire_store(base_b + k, rows_b.at[pl.ds(k * CAT_CHUNK, CAT_CHUNK)],
                       ss_b)

    for k in range(k_per_set):
        wait_store(rows_a, k, ss_a)
        wait_store(rows_b, k, ss_b)


def _sc_body(B, table_ref, src_idx_ref, dst_cat_ref, num_ref, dst_num_ref,
             static_src_ref, dst_static_ref, out_ref,
             cat_src_v, cat_dst_v, num_dst_v, static_dst_v, static_rows,
             rows_a, rows_b, gs_a, gs_b, ss_a, ss_b, st_sem):
    per_w = B // NW
    n_cat_chunks = per_w * N_CAT // CAT_CHUNK
    n_num_chunks = per_w * N_NUM // NUM_CHUNK
    n_static_chunks = per_w // STATIC_GROUP

    w = lax.axis_index("s") * NC + lax.axis_index("c")

    pltpu.sync_copy(src_idx_ref.at[w], cat_src_v)
    pltpu.sync_copy(dst_cat_ref.at[w], cat_dst_v)
    pltpu.sync_copy(dst_num_ref.at[w], num_dst_v)
    pltpu.sync_copy(dst_static_ref.at[w], static_dst_v)
    pltpu.sync_copy(static_src_ref, static_rows)

    def cat_load(chunk, rows_slice, sem):
        return pltpu.async_copy(table_ref.at[cat_src_v.at[chunk]], rows_slice,
                                sem)

    def cat_store(chunk, rows_slice, sem):
        return pltpu.async_copy(rows_slice, out_ref.at[cat_dst_v.at[chunk]],
                                sem)

    _pipelined_phase(n_cat_chunks, KSET, rows_a, rows_b, gs_a, gs_b,
                     ss_a, ss_b, out_ref, cat_dst_v.at[0],
                     cat_load, cat_store)

    num_base = w * per_w * N_NUM

    def num_load(chunk, rows_slice, sem):
        return pltpu.async_copy(
            num_ref.at[pl.ds(num_base + chunk * NUM_CHUNK, NUM_CHUNK)],
            rows_slice, sem)

    def num_store(chunk, rows_slice, sem):
        return pltpu.async_copy(rows_slice, out_ref.at[num_dst_v.at[chunk]],
                                sem)

    _pipelined_phase(n_num_chunks, 2, rows_a, rows_b, gs_a, gs_b,
                     ss_a, ss_b, out_ref, num_dst_v.at[0],
                     num_load, num_store)

    @pl.loop(0, n_static_chunks)
    def _static_fire(g):
        pltpu.async_copy(static_rows, out_ref.at[static_dst_v.at[g]], st_sem)

    @pl.loop(0, n_static_chunks)
    def _static_drain(g):
        pltpu.make_async_copy(static_rows, out_ref.at[static_dst_v.at[0]],
                              st_sem).wait()


def _sc_assemble(B, table, src_idx, dst_cat, num_emb, dst_num, static_src,
                 dst_static):
    mesh = plsc.VectorSubcoreMesh(core_axis_name="c", subcore_axis_name="s")
    per_w = B // NW
    static_len = STATIC_GROUP * STATIC_ROWS
    kern = pl.kernel(
        functools.partial(_sc_body, B),
        out_type=jax.ShapeDtypeStruct((B * ROWS_PER_B, D), jnp.float32),
        mesh=mesh,
        scratch_types=[
            pltpu.VMEM((per_w * N_CAT // CAT_CHUNK, CAT_CHUNK), jnp.int32),
            pltpu.VMEM((per_w * N_CAT // CAT_CHUNK, CAT_CHUNK), jnp.int32),
            pltpu.VMEM((per_w * N_NUM // NUM_CHUNK, NUM_CHUNK), jnp.int32),
            pltpu.VMEM((per_w // STATIC_GROUP, static_len), jnp.int32),
            pltpu.VMEM((static_len, D), jnp.float32),
            pltpu.VMEM((KSET * CAT_CHUNK, D), jnp.float32),
            pltpu.VMEM((KSET * CAT_CHUNK, D), jnp.float32),
            pltpu.SemaphoreType.DMA,
            pltpu.SemaphoreType.DMA,
            pltpu.SemaphoreType.DMA,
            pltpu.SemaphoreType.DMA,
            pltpu.SemaphoreType.DMA,
        ],
        compiler_params=pltpu.CompilerParams(use_tc_tiling_on_sc=False),
    )
    return kern(table, src_idx, dst_cat, num_emb, dst_num, static_src,
                dst_static)


def kernel(x_cat, x_num, cat_tables, num_w1, num_b1, num_w2, num_b2,
           pos_table, cls_token):
    B = x_cat.shape[0]
    per_w = B // NW

    table = cat_tables.reshape(N_CAT * VOCAB, D)
    b = jnp.arange(B, dtype=jnp.int32)
    f = jnp.arange(N_CAT, dtype=jnp.int32)
    j = jnp.arange(N_NUM, dtype=jnp.int32)

    src_idx = (x_cat + (f * VOCAB)[None, :]).reshape(
        NW, per_w * N_CAT // CAT_CHUNK, CAT_CHUNK)
    dst_cat = (ROWS_PER_B * b[:, None] + 2 * f[None, :]).reshape(
        NW, per_w * N_CAT // CAT_CHUNK, CAT_CHUNK)
    dst_num = (ROWS_PER_B * b[:, None] + 2 * N_CAT + 2 * j[None, :]).reshape(
        NW, per_w * N_NUM // NUM_CHUNK, NUM_CHUNK)

    offs_static = jnp.concatenate([
        2 * jnp.arange(T, dtype=jnp.int32) + 1,
        jnp.array([2 * T, 2 * T + 1], dtype=jnp.int32),
    ])
    dst_static = (ROWS_PER_B * b[:, None] + offs_static[None, :]).reshape(
        NW, per_w // STATIC_GROUP, STATIC_GROUP * STATIC_ROWS)

    cls = cls_token.reshape(2 * D)
    static_pat = jnp.concatenate(
        [pos_table, cls[:D][None, :], cls[D:][None, :]], axis=0)
    static_src = jnp.tile(static_pat, (STATIC_GROUP, 1))

    num_w2t = num_w2.transpose(0, 2, 1)

    num_emb = _num_mlp(x_num, num_w1, num_b1, num_w2t, num_b2)
    num_rows = num_emb.reshape(B * N_NUM, D)

    out_flat = _sc_assemble(B, table, src_idx, dst_cat, num_rows, dst_num,
                            static_src, dst_static)
    return out_flat.reshape(B, T + 1, 2 * D)


# speedup vs baseline: 1.3552x; 1.1175x over previous
import functools

import jax
import jax.numpy as jnp
from jax import lax
from jax.experimental import pallas as pl
from jax.experimental.pallas import tpu as pltpu
from jax.experimental.pallas import tpu_sc as plsc

N_CAT = 26
N_NUM = 13
VOCAB = 100000
D = 32
T = N_CAT + N_NUM
TP1 = T + 1
PHYS = 128

NC = 2
NS = 16
NW = NC * NS

G = 4
NUM_ROWS = G * N_NUM * D // PHYS


def _num_mlp_body(x_ref, w1_ref, b1_ref, w2t_ref, b2_ref, out_ref):
    x = x_ref[...]
    outs = []
    for n in range(N_NUM):
        h1 = x[:, n:n + 1] * w1_ref[n][None, :] + b1_ref[n][None, :]
        z = jax.lax.dot_general(
            h1, w2t_ref[n],
            dimension_numbers=(((1,), (0,)), ((), ())),
            preferred_element_type=jnp.float32,
        ) + b2_ref[n][None, :]
        outs.append(jnp.tanh(z))
    out_ref[...] = jnp.concatenate(outs, axis=1)


def _num_mlp(x_num, num_w1, num_b1, num_w2t, num_b2):
    B = x_num.shape[0]
    BB = 2048
    grid = (B // BB,)
    return pl.pallas_call(
        _num_mlp_body,
        grid=grid,
        in_specs=[
            pl.BlockSpec((BB, N_NUM), lambda i: (i, 0)),
            pl.BlockSpec((N_NUM, D), lambda i: (0, 0)),
            pl.BlockSpec((N_NUM, D), lambda i: (0, 0)),
            pl.BlockSpec((N_NUM, D, D), lambda i: (0, 0, 0)),
            pl.BlockSpec((N_NUM, D), lambda i: (0, 0)),
        ],
        out_specs=pl.BlockSpec((BB, N_NUM * D), lambda i: (i, 0)),
        out_shape=jax.ShapeDtypeStruct((B, N_NUM * D), jnp.float32),
    )(x_num, num_w1, num_b1, num_w2t, num_b2)


def _fill(buf, cat, num):
    for g in range(G):
        for r in range(N_CAT):
            src = g * N_CAT + r
            dst = g * TP1 + r
            buf[dst, 0:16] = cat[src, 0:16]
            buf[dst, 16:32] = cat[src, 16:32]
        for j in range(N_NUM):
            dst = g * TP1 + N_CAT + j
            buf[dst, 0:16] = num[g, j * D:j * D + 16]
            buf[dst, 16:32] = num[g, j * D + 16:j * D + 32]


def _sc_body(B, table_ref, src_idx_ref, num_ref, tmpl_ref, out_ref,
             idx_v, buf_a, buf_b, cat_a, cat_b, num_a, num_b,
             gs_a, gs_b, ss_a, ss_b):
    per_w = B // NW
    chunks = per_w // G
    rows_pc = G * TP1

    w = lax.axis_index("s") * NC + lax.axis_index("c")
    wrow0 = w * per_w * TP1
    nrow0 = w * per_w

    pltpu.sync_copy(src_idx_ref.at[w], idx_v)
    for s in range(G):
        pltpu.sync_copy(tmpl_ref, buf_a.at[pl.ds(s * TP1, TP1)])
        pltpu.sync_copy(tmpl_ref, buf_b.at[pl.ds(s * TP1, TP1)])

    def fire_set(c, cat, num, gsem):
        d1 = pltpu.async_copy(table_ref.at[idx_v.at[c]], cat, gsem)
        d2 = pltpu.async_copy(num_ref.at[pl.ds(nrow0 + c * G, G)], num, gsem)
        return d1, d2

    def fire_scatter(c, buf, ssem):
        return pltpu.async_copy(
            buf, out_ref.at[pl.ds(wrow0 + c * rows_pc, rows_pc)], ssem)

    def drain_scatter(buf, ssem):
        pltpu.make_async_copy(
            buf, out_ref.at[pl.ds(wrow0, rows_pc)], ssem).wait()

    @pl.loop(0, chunks, step=2)
    def _pair(c):
        da = fire_set(c, cat_a, num_a, gs_a)
        db = fire_set(c + 1, cat_b, num_b, gs_b)

        da[0].wait()
        da[1].wait()

        @pl.when(c > 0)
        def _():
            drain_scatter(buf_a, ss_a)

        _fill(buf_a, cat_a, num_a)
        fire_scatter(c, buf_a, ss_a)

        db[0].wait()
        db[1].wait()

        @pl.when(c > 0)
        def _():
            drain_scatter(buf_b, ss_b)

        _fill(buf_b, cat_b, num_b)
        fire_scatter(c + 1, buf_b, ss_b)

    drain_scatter(buf_a, ss_a)
    drain_scatter(buf_b, ss_b)


def _sc_assemble(B, table, src_idx, num_dense, tmpl):
    mesh = plsc.VectorSubcoreMesh(core_axis_name="c", subcore_axis_name="s")
    per_w = B // NW
    chunks = per_w // G
    kern = pl.kernel(
        functools.partial(_sc_body, B),
        out_type=jax.ShapeDtypeStruct((B * TP1, PHYS), jnp.float32),
        mesh=mesh,
        scratch_types=[
            pltpu.VMEM((chunks, G * N_CAT), jnp.int32),
            pltpu.VMEM((G * TP1, PHYS), jnp.float32),
            pltpu.VMEM((G * TP1, PHYS), jnp.float32),
            pltpu.VMEM((G * N_CAT, D), jnp.float32),
            pltpu.VMEM((G * N_CAT, D), jnp.float32),
            pltpu.VMEM((G, N_NUM * D), jnp.float32),
            pltpu.VMEM((G, N_NUM * D), jnp.float32),
            pltpu.SemaphoreType.DMA,
            pltpu.SemaphoreType.DMA,
            pltpu.SemaphoreType.DMA,
            pltpu.SemaphoreType.DMA,
        ],
        compiler_params=pltpu.CompilerParams(use_tc_tiling_on_sc=False),
    )
    return kern(table, src_idx, num_dense, tmpl)


def kernel(x_cat, x_num, cat_tables, num_w1, num_b1, num_w2, num_b2,
           pos_table, cls_token):
    B = x_cat.shape[0]
    per_w = B // NW
    chunks = per_w // G

    table = cat_tables.reshape(N_CAT * VOCAB, D)
    f = jnp.arange(N_CAT, dtype=jnp.int32)
    src_idx = (x_cat + (f * VOCAB)[None, :]).reshape(NW, chunks, G * N_CAT)

    cls = cls_token.reshape(2 * D)
    tmpl = jnp.zeros((TP1, PHYS), jnp.float32)
    tmpl = tmpl.at[:T, D:2 * D].set(pos_table)
    tmpl = tmpl.at[T, :2 * D].set(cls)

    num_w2t = num_w2.transpose(0, 2, 1)

    num_dense = _num_mlp(x_num, num_w1, num_b1, num_w2t, num_b2)
    out_phys = _sc_assemble(B, table, src_idx, num_dense, tmpl)
    return out_phys[:, :2 * D].reshape(B, TP1, 2 * D)


# speedup vs baseline: 3.6067x; 2.6614x over previous
import jax
import jax.numpy as jnp
from jax import lax
from jax.experimental import pallas as pl
from jax.experimental.pallas import tpu as pltpu
from jax.experimental.pallas import tpu_sc as plsc

N_CAT = 26
N_NUM = 13
VOCAB = 100000
D = 32
T = N_CAT + N_NUM
TP1 = T + 1
W2 = 2 * D

NC = 2
NS = 16
NW = NC * NS

CAT_PW = N_CAT * D // NW
NUM_PW = N_NUM * D // NW
N_STATIC = T * D + W2
STA_PW = N_STATIC // NW

SUB = 4096
NSUB = 4


def _num_mlp_body(xt_ref, w1_ref, b1_ref, w2_ref, b2_ref, out_ref):
    for n in range(N_NUM):
        h1t = (w1_ref[n][:, None] * xt_ref[n][None, :]
               + b1_ref[n][:, None])
        z = jax.lax.dot_general(
            w2_ref[n], h1t,
            dimension_numbers=(((1,), (0,)), ((), ())),
            preferred_element_type=jnp.float32,
        ) + b2_ref[n][:, None]
        out_ref[n] = jnp.tanh(z)


def _num_mlp(xt, num_w1, num_b1, num_w2, num_b2):
    B = xt.shape[1]
    BB = 2048
    grid = (B // BB,)
    return pl.pallas_call(
        _num_mlp_body,
        grid=grid,
        in_specs=[
            pl.BlockSpec((N_NUM, BB), lambda i: (0, i)),
            pl.BlockSpec((N_NUM, D), lambda i: (0, 0)),
            pl.BlockSpec((N_NUM, D), lambda i: (0, 0)),
            pl.BlockSpec((N_NUM, D, D), lambda i: (0, 0, 0)),
            pl.BlockSpec((N_NUM, D), lambda i: (0, 0)),
        ],
        out_specs=pl.BlockSpec((N_NUM, D, BB), lambda i: (0, 0, i)),
        out_shape=jax.ShapeDtypeStruct((N_NUM, D, B), jnp.float32),
    )(xt, num_w1, num_b1, num_w2, num_b2)


def _sc_body(table_ref, xcat_ref, num_ref, const_ref, out_ref,
             col, idxb, pb, cvm, csem, wsem):
    w = lax.axis_index("s") * NC + lax.axis_index("c")

    pltpu.sync_copy(const_ref, cvm)

    def out_at(t, c, h):
        return out_ref.at[t, c, pl.ds(h * SUB, SUB)]

    def static_plane(s):
        is_cls = s >= T * D
        t = jnp.where(is_cls, T, s // D)
        c = jnp.where(is_cls, s - T * D, D + s % D)
        iv = jnp.full((16,), t * W2 + c, jnp.int32)
        vec = plsc.load_gather(cvm, [iv])

        @pl.loop(0, SUB // 16)
        def _(i):
            pb[0, pl.ds(i * 16, 16)] = vec

        for h in range(NSUB):
            pltpu.sync_copy(pb.at[0], out_at(t, c, h))

    def num_plane(q):
        j = q // D
        c = q % D
        for h in range(NSUB):
            pltpu.sync_copy(num_ref.at[j, c, pl.ds(h * SUB, SUB)], pb.at[1])
            pltpu.sync_copy(pb.at[1], out_at(N_CAT + j, c, h))

    @pl.loop(0, CAT_PW)
    def _cat(k):
        p = w * CAT_PW + k
        f = p // D
        c = p % D

        cd = pltpu.async_copy(table_ref.at[p], col, csem)

        @pl.when(k > 0)
        def _():
            for _i in range(2):
                pltpu.make_async_copy(pb.at[0], out_at(0, 0, 0), wsem).wait()

        @pl.when(k < NUM_PW)
        def _():
            num_plane(w * NUM_PW + k)

        static_plane(w * STA_PW + k)

        @pl.when(k < STA_PW - CAT_PW)
        def _():
            static_plane(w * STA_PW + CAT_PW + k)

        cd.wait()

        for h in range(NSUB):
            bi = h % 2
            pltpu.sync_copy(xcat_ref.at[f, pl.ds(h * SUB, SUB)], idxb)
            if h >= 2:
                pltpu.make_async_copy(pb.at[0], out_at(0, 0, 0), wsem).wait()

            @pl.loop(0, SUB // 16)
            def _(i):
                iv = idxb[pl.ds(i * 16, 16)]
                pb[bi, pl.ds(i * 16, 16)] = plsc.load_gather(col, [iv])

            pltpu.async_copy(pb.at[bi], out_at(f, c, h), wsem)

    for _i in range(2):
        pltpu.make_async_copy(pb.at[0], out_ref.at[0, 0, pl.ds(0, SUB)],
                              wsem).wait()


def _sc_assemble(table2, xcat_t, num_planes, consts):
    B = xcat_t.shape[1]
    mesh = plsc.VectorSubcoreMesh(core_axis_name="c", subcore_axis_name="s")
    kern = pl.kernel(
        _sc_body,
        out_type=jax.ShapeDtypeStruct((TP1, W2, B), jnp.float32),
        mesh=mesh,
        scratch_types=[
            pltpu.VMEM((VOCAB,), jnp.float32),
            pltpu.VMEM((SUB,), jnp.int32),
            pltpu.VMEM((2, SUB), jnp.float32),
            pltpu.VMEM((TP1 * W2,), jnp.float32),
            pltpu.SemaphoreType.DMA,
            pltpu.SemaphoreType.DMA,
        ],
        compiler_params=pltpu.CompilerParams(use_tc_tiling_on_sc=True,
                                             needs_layout_passes=False),
    )
    return kern(table2, xcat_t, num_planes, consts)


def kernel(x_cat, x_num, cat_tables, num_w1, num_b1, num_w2, num_b2,
           pos_table, cls_token):
    B = x_cat.shape[0]

    table2 = cat_tables.transpose(0, 2, 1).reshape(N_CAT * D, VOCAB)
    xcat_t = x_cat.T
    xt = x_num.T

    cls = cls_token.reshape(W2)
    consts = jnp.zeros((TP1, W2), jnp.float32)
    consts = consts.at[:T, D:].set(pos_table)
    consts = consts.at[T, :].set(cls)
    consts = consts.reshape(TP1 * W2)

    num_planes = _num_mlp(xt, num_w1, num_b1, num_w2, num_b2)
    out_phys = _sc_assemble(table2, xcat_t, num_planes, consts)
    return out_phys.transpose(2, 0, 1)


# speedup vs baseline: 3.8372x; 1.0639x over previous
import jax
import jax.numpy as jnp
from jax import lax
from jax.experimental import pallas as pl
from jax.experimental.pallas import tpu as pltpu
from jax.experimental.pallas import tpu_sc as plsc

N_CAT = 26
N_NUM = 13
VOCAB = 100000
D = 32
T = N_CAT + N_NUM
TP1 = T + 1
W2 = 2 * D

NC = 2
NS = 16
NW = NC * NS

CAT_PW = N_CAT * D // NW
NUM_PW = N_NUM * D // NW
N_STATIC = T * D + W2
STA_PW = N_STATIC // NW

SUB = 4096
SSUB = 2048
NSUB = 4


def _num_mlp_body(xt_ref, w1_ref, b1_ref, w2_ref, b2_ref, out_ref):
    for n in range(N_NUM):
        h1t = (w1_ref[n][:, None] * xt_ref[n][None, :]
               + b1_ref[n][:, None])
        z = jax.lax.dot_general(
            w2_ref[n], h1t,
            dimension_numbers=(((1,), (0,)), ((), ())),
            preferred_element_type=jnp.float32,
        ) + b2_ref[n][:, None]
        out_ref[n] = jnp.tanh(z)


def _num_mlp(xt, num_w1, num_b1, num_w2, num_b2):
    B = xt.shape[1]
    BB = 2048
    grid = (B // BB,)
    return pl.pallas_call(
        _num_mlp_body,
        grid=grid,
        in_specs=[
            pl.BlockSpec((N_NUM, BB), lambda i: (0, i)),
            pl.BlockSpec((N_NUM, D), lambda i: (0, 0)),
            pl.BlockSpec((N_NUM, D), lambda i: (0, 0)),
            pl.BlockSpec((N_NUM, D, D), lambda i: (0, 0, 0)),
            pl.BlockSpec((N_NUM, D), lambda i: (0, 0)),
        ],
        out_specs=pl.BlockSpec((N_NUM, D, BB), lambda i: (0, 0, i)),
        out_shape=jax.ShapeDtypeStruct((N_NUM, D, B), jnp.float32),
    )(xt, num_w1, num_b1, num_w2, num_b2)


HALF = 50048
HB0 = 49920
HB_LEN = VOCAB - HB0


def _sc_body(table_ref, xcat_ref, num_ref, const_ref, out_ref,
             col_a, col_b, res0, res1, res2, idx0, idx1, sbuf, nbuf, cvm,
             casem, cbsem, wsem, ssem, nsem):
    res_l = [res0, res1, res2]
    idx_l = [idx0, idx1]
    w = lax.axis_index("s") * NC + lax.axis_index("c")

    pltpu.sync_copy(const_ref, cvm)

    def out_at(t, c, h):
        return out_ref.at[t, c, pl.ds(h * SUB, SUB)]

    def drain(n, buf, sem):
        for _ in range(n):
            pltpu.make_async_copy(buf, out_at(0, 0, 0), sem).wait()

    def static_plane(s, first):
        @pl.when(jnp.logical_not(first))
        def _():
            for _ in range(2 * NSUB):
                pltpu.make_async_copy(
                    sbuf, out_ref.at[0, 0, pl.ds(0, SSUB)], ssem).wait()

        is_cls = s >= T * D
        t = jnp.where(is_cls, T, s // D)
        c = jnp.where(is_cls, s - T * D, D + s % D)
        iv = jnp.full((16,), t * W2 + c, jnp.int32)
        vec = plsc.load_gather(cvm, [iv])

        @pl.loop(0, SSUB // 16)
        def _(i):
            sbuf[pl.ds(i * 16, 16)] = vec

        for h in range(2 * NSUB):
            pltpu.async_copy(
                sbuf, out_ref.at[t, c, pl.ds(h * SSUB, SSUB)], ssem)

    def num_plane(q, first):
        j = q // D
        c = q % D
        for h in range(NSUB):
            if h > 0:
                drain(1, nbuf, nsem)
            else:
                @pl.when(jnp.logical_not(first))
                def _():
                    drain(1, nbuf, nsem)
            pltpu.sync_copy(num_ref.at[j, c, pl.ds(h * SUB, SUB)], nbuf)
            pltpu.async_copy(nbuf, out_at(N_CAT + j, c, h), nsem)

    pltpu.async_copy(table_ref.at[pl.ds(w * CAT_PW, 1), pl.ds(0, HALF)],
                     col_a, casem)

    @pl.loop(0, CAT_PW)
    def _cat(k):
        p = w * CAT_PW + k
        f = p // D
        c = p % D

        cb = pltpu.async_copy(table_ref.at[pl.ds(p, 1), pl.ds(HB0, HB_LEN)],
                              col_b, cbsem)

        @pl.when(k < NUM_PW)
        def _():
            num_plane(w * NUM_PW + k, k == 0)

        static_plane(w * STA_PW + k, k == 0)

        @pl.when(k < STA_PW - CAT_PW)
        def _():
            static_plane(w * STA_PW + CAT_PW + k, False)

        pltpu.make_async_copy(
            table_ref.at[pl.ds(p, 1), pl.ds(0, HALF)], col_a, casem).wait()

        def gather_lo(ii, slot):
            idx_r, res_r = idx_l[ii], res_l[slot]

            @pl.loop(0, SUB // 16)
            def _(i):
                iv = idx_r[pl.ds(i * 16, 16)]
                iv_a = jnp.minimum(iv, HALF - 1)
                iv0 = jnp.zeros((16,), jnp.int32)
                res_r[pl.ds(i * 16, 16)] = plsc.load_gather(col_a,
                                                            [iv0, iv_a])

        def gather_hi(ii, slot):
            idx_r, res_r = idx_l[ii], res_l[slot]

            @pl.loop(0, SUB // 16)
            def _(i):
                iv = idx_r[pl.ds(i * 16, 16)]
                m = iv >= HB0
                iv_b = jnp.maximum(iv - HB0, 0)
                iv0 = jnp.zeros((16,), jnp.int32)
                g_b = plsc.load_gather(col_b, [iv0, iv_b])
                cur = res_r[pl.ds(i * 16, 16)]
                res_r[pl.ds(i * 16, 16)] = jnp.where(m, g_b, cur)

        for g2 in range(2):
            h0, h1 = 2 * g2, 2 * g2 + 1
            s0, s1 = (0, 1) if g2 == 0 else (2, 0)
            pltpu.sync_copy(xcat_ref.at[f, pl.ds(h0 * SUB, SUB)], idx0)
            pltpu.sync_copy(xcat_ref.at[f, pl.ds(h1 * SUB, SUB)], idx1)

            if g2 == 0:
                @pl.when(k > 0)
                def _():
                    drain(2, res0, wsem)
            else:
                drain(2, res0, wsem)

            gather_lo(0, s0)
            gather_lo(1, s1)

            if g2 == 0:
                cb.wait()
            else:
                @pl.when(k + 1 < CAT_PW)
                def _():
                    pltpu.async_copy(
                        table_ref.at[pl.ds(p + 1, 1), pl.ds(0, HALF)],
                        col_a, casem)

            gather_hi(0, s0)
            gather_hi(1, s1)

            pltpu.async_copy(res_l[s0], out_at(f, c, h0), wsem)
            pltpu.async_copy(res_l[s1], out_at(f, c, h1), wsem)

    drain(2, res0, wsem)
    for _ in range(2 * NSUB):
        pltpu.make_async_copy(sbuf, out_ref.at[0, 0, pl.ds(0, SSUB)],
                              ssem).wait()
    drain(1, nbuf, nsem)


def _sc_assemble(table2, xcat_t, num_planes, consts):
    B = xcat_t.shape[1]
    mesh = plsc.VectorSubcoreMesh(core_axis_name="c", subcore_axis_name="s")
    kern = pl.kernel(
        _sc_body,
        out_type=jax.ShapeDtypeStruct((TP1, W2, B), jnp.float32),
        mesh=mesh,
        scratch_types=[
            pltpu.VMEM((1, HALF), jnp.float32),
            pltpu.VMEM((1, HB_LEN), jnp.float32),
            pltpu.VMEM((SUB,), jnp.float32),
            pltpu.VMEM((SUB,), jnp.float32),
            pltpu.VMEM((SUB,), jnp.float32),
            pltpu.VMEM((SUB,), jnp.int32),
            pltpu.VMEM((SUB,), jnp.int32),
            pltpu.VMEM((SSUB,), jnp.float32),
            pltpu.VMEM((SUB,), jnp.float32),
            pltpu.VMEM((TP1 * W2,), jnp.float32),
            pltpu.SemaphoreType.DMA,
            pltpu.SemaphoreType.DMA,
            pltpu.SemaphoreType.DMA,
            pltpu.SemaphoreType.DMA,
            pltpu.SemaphoreType.DMA,
        ],
        compiler_params=pltpu.CompilerParams(use_tc_tiling_on_sc=True,
                                             needs_layout_passes=False),
    )
    return kern(table2, xcat_t, num_planes, consts)


def kernel(x_cat, x_num, cat_tables, num_w1, num_b1, num_w2, num_b2,
           pos_table, cls_token):
    B = x_cat.shape[0]

    table2 = cat_tables.transpose(0, 2, 1).reshape(N_CAT * D, VOCAB)
    xcat_t = x_cat.T
    xt = x_num.T

    cls = cls_token.reshape(W2)
    consts = jnp.zeros((TP1, W2), jnp.float32)
    consts = consts.at[:T, D:].set(pos_table)
    consts = consts.at[T, :].set(cls)
    consts = consts.reshape(TP1 * W2)

    num_planes = _num_mlp(xt, num_w1, num_b1, num_w2, num_b2)
    out_phys = _sc_assemble(table2, xcat_t, num_planes, consts)
    return out_phys.transpose(2, 0, 1)


# speedup vs baseline: 4.2494x; 1.1074x over previous
import jax
import jax.numpy as jnp
from jax import lax
from jax.experimental import pallas as pl
from jax.experimental.pallas import tpu as pltpu
from jax.experimental.pallas import tpu_sc as plsc

N_CAT = 26
N_NUM = 13
VOCAB = 100000
D = 32
T = N_CAT + N_NUM
TP1 = T + 1
W2 = 2 * D

NC = 2
NS = 16
NW = NC * NS

CAT_PW = N_CAT * D // NW
NUM_PW = N_NUM * D // NW
N_STATIC = T * D + W2
STA_PW = N_STATIC // NW

SUB = 4096
SSUB = 2048
NSUB = 4


def _num_mlp_body(xt_ref, w1_ref, b1_ref, w2_ref, b2_ref, out_ref):
    for n in range(N_NUM):
        h1t = (w1_ref[n][:, None] * xt_ref[n][None, :]
               + b1_ref[n][:, None])
        z = jax.lax.dot_general(
            w2_ref[n], h1t,
            dimension_numbers=(((1,), (0,)), ((), ())),
            preferred_element_type=jnp.float32,
        ) + b2_ref[n][:, None]
        out_ref[n] = jnp.tanh(z)


def _num_mlp(xt, num_w1, num_b1, num_w2, num_b2):
    B = xt.shape[1]
    BB = 2048
    grid = (B // BB,)
    return pl.pallas_call(
        _num_mlp_body,
        grid=grid,
        in_specs=[
            pl.BlockSpec((N_NUM, BB), lambda i: (0, i)),
            pl.BlockSpec((N_NUM, D), lambda i: (0, 0)),
            pl.BlockSpec((N_NUM, D), lambda i: (0, 0)),
            pl.BlockSpec((N_NUM, D, D), lambda i: (0, 0, 0)),
            pl.BlockSpec((N_NUM, D), lambda i: (0, 0)),
        ],
        out_specs=pl.BlockSpec((N_NUM, D, BB), lambda i: (0, 0, i)),
        out_shape=jax.ShapeDtypeStruct((N_NUM, D, B), jnp.float32),
    )(xt, num_w1, num_b1, num_w2, num_b2)


HALF = 50048
HB0 = 49920
HB_LEN = VOCAB - HB0


def _sc_body(table_ref, xcat_ref, num_ref, const_ref, out_ref,
             col_a, col_b, res0, res1, res2, idx0, idx1, sbuf, nbuf, cvm,
             casem, cbsem, wsem, ssem, nsem, isem):
    res_l = [res0, res1, res2]
    idx_l = [idx0, idx1]
    w = lax.axis_index("s") * NC + lax.axis_index("c")

    pltpu.sync_copy(const_ref, cvm)

    def out_at(t, c, h):
        return out_ref.at[t, c, pl.ds(h * SUB, SUB)]

    def drain(n, buf, sem):
        for _ in range(n):
            pltpu.make_async_copy(buf, out_at(0, 0, 0), sem).wait()

    def static_plane(s, first):
        @pl.when(jnp.logical_not(first))
        def _():
            for _ in range(2 * NSUB):
                pltpu.make_async_copy(
                    sbuf, out_ref.at[0, 0, pl.ds(0, SSUB)], ssem).wait()

        is_cls = s >= T * D
        t = jnp.where(is_cls, T, s // D)
        c = jnp.where(is_cls, s - T * D, D + s % D)
        iv = jnp.full((16,), t * W2 + c, jnp.int32)
        vec = plsc.load_gather(cvm, [iv])

        @pl.loop(0, SSUB // 16)
        def _(i):
            sbuf[pl.ds(i * 16, 16)] = vec

        for h in range(2 * NSUB):
            pltpu.async_copy(
                sbuf, out_ref.at[t, c, pl.ds(h * SSUB, SSUB)], ssem)

    def num_plane(q, first):
        j = q // D
        c = q % D
        for h in range(NSUB):
            if h > 0:
                drain(1, nbuf, nsem)
            else:
                @pl.when(jnp.logical_not(first))
                def _():
                    drain(1, nbuf, nsem)
            pltpu.sync_copy(num_ref.at[j, c, pl.ds(h * SUB, SUB)], nbuf)
            pltpu.async_copy(nbuf, out_at(N_CAT + j, c, h), nsem)

    pltpu.async_copy(table_ref.at[pl.ds(w * CAT_PW, 1), pl.ds(0, HALF)],
                     col_a, casem)

    @pl.loop(0, CAT_PW)
    def _cat(k):
        p = w * CAT_PW + k
        f = p // D
        c = p % D

        cb = pltpu.async_copy(table_ref.at[pl.ds(p, 1), pl.ds(HB0, HB_LEN)],
                              col_b, cbsem)

        @pl.when(k < NUM_PW)
        def _():
            num_plane(w * NUM_PW + k, k == 0)

        static_plane(w * STA_PW + k, k == 0)

        @pl.when(k < STA_PW - CAT_PW)
        def _():
            static_plane(w * STA_PW + CAT_PW + k, False)

        pltpu.make_async_copy(
            table_ref.at[pl.ds(p, 1), pl.ds(0, HALF)], col_a, casem).wait()

        def gather_lo(ii, slot):
            idx_r, res_r = idx_l[ii], res_l[slot]

            @pl.loop(0, SUB // 64)
            def _(i):
                for u in range(4):
                    o = i * 64 + u * 16
                    iv = idx_r[pl.ds(o, 16)]
                    iv_a = jnp.minimum(iv, HALF - 1)
                    iv0 = jnp.zeros((16,), jnp.int32)
                    res_r[pl.ds(o, 16)] = plsc.load_gather(col_a,
                                                           [iv0, iv_a])

        def gather_hi(ii, slot):
            idx_r, res_r = idx_l[ii], res_l[slot]

            @pl.loop(0, SUB // 64)
            def _(i):
                for u in range(4):
                    o = i * 64 + u * 16
                    iv = idx_r[pl.ds(o, 16)]
                    m = iv >= HB0
                    iv_b = jnp.maximum(iv - HB0, 0)
                    iv0 = jnp.zeros((16,), jnp.int32)
                    g_b = plsc.load_gather(col_b, [iv0, iv_b])
                    cur = res_r[pl.ds(o, 16)]
                    res_r[pl.ds(o, 16)] = jnp.where(m, g_b, cur)

        for g2 in range(2):
            h0, h1 = 2 * g2, 2 * g2 + 1
            s0, s1 = (0, 1) if g2 == 0 else (2, 0)
            di0 = pltpu.async_copy(xcat_ref.at[f, pl.ds(h0 * SUB, SUB)],
                                   idx0, isem)
            di1 = pltpu.async_copy(xcat_ref.at[f, pl.ds(h1 * SUB, SUB)],
                                   idx1, isem)

            if g2 == 0:
                @pl.when(k > 0)
                def _():
                    drain(2, res0, wsem)
            else:
                drain(2, res0, wsem)

            di0.wait()
            gather_lo(0, s0)
            di1.wait()
            gather_lo(1, s1)

            if g2 == 0:
                cb.wait()
            else:
                @pl.when(k + 1 < CAT_PW)
                def _():
                    pltpu.async_copy(
                        table_ref.at[pl.ds(p + 1, 1), pl.ds(0, HALF)],
                        col_a, casem)

            gather_hi(0, s0)
            gather_hi(1, s1)

            pltpu.async_copy(res_l[s0], out_at(f, c, h0), wsem)
            pltpu.async_copy(res_l[s1], out_at(f, c, h1), wsem)

    drain(2, res0, wsem)
    for _ in range(2 * NSUB):
        pltpu.make_async_copy(sbuf, out_ref.at[0, 0, pl.ds(0, SSUB)],
                              ssem).wait()
    drain(1, nbuf, nsem)


def _sc_assemble(table2, xcat_t, num_planes, consts):
    B = xcat_t.shape[1]
    mesh = plsc.VectorSubcoreMesh(core_axis_name="c", subcore_axis_name="s")
    kern = pl.kernel(
        _sc_body,
        out_type=jax.ShapeDtypeStruct((TP1, W2, B), jnp.float32),
        mesh=mesh,
        scratch_types=[
            pltpu.VMEM((1, HALF), jnp.float32),
            pltpu.VMEM((1, HB_LEN), jnp.float32),
            pltpu.VMEM((SUB,), jnp.float32),
            pltpu.VMEM((SUB,), jnp.float32),
            pltpu.VMEM((SUB,), jnp.float32),
            pltpu.VMEM((SUB,), jnp.int32),
            pltpu.VMEM((SUB,), jnp.int32),
            pltpu.VMEM((SSUB,), jnp.float32),
            pltpu.VMEM((SUB,), jnp.float32),
            pltpu.VMEM((TP1 * W2,), jnp.float32),
            pltpu.SemaphoreType.DMA,
            pltpu.SemaphoreType.DMA,
            pltpu.SemaphoreType.DMA,
            pltpu.SemaphoreType.DMA,
            pltpu.SemaphoreType.DMA,
            pltpu.SemaphoreType.DMA,
        ],
        compiler_params=pltpu.CompilerParams(use_tc_tiling_on_sc=True,
                                             needs_layout_passes=False),
    )
    return kern(table2, xcat_t, num_planes, consts)


def kernel(x_cat, x_num, cat_tables, num_w1, num_b1, num_w2, num_b2,
           pos_table, cls_token):
    B = x_cat.shape[0]

    table2 = cat_tables.transpose(0, 2, 1).reshape(N_CAT * D, VOCAB)
    xcat_t = x_cat.T
    xt = x_num.T

    cls = cls_token.reshape(W2)
    consts = jnp.zeros((TP1, W2), jnp.float32)
    consts = consts.at[:T, D:].set(pos_table)
    consts = consts.at[T, :].set(cls)
    consts = consts.reshape(TP1 * W2)

    num_planes = _num_mlp(xt, num_w1, num_b1, num_w2, num_b2)
    out_phys = _sc_assemble(table2, xcat_t, num_planes, consts)
    return out_phys.transpose(2, 0, 1)
